# baseline (speedup 1.0000x reference)
import functools

import jax
import jax.numpy as jnp
from jax import lax
from jax.experimental import pallas as pl
from jax.experimental.pallas import tpu as pltpu
from jax.experimental.pallas import tpu_sc as plsc

D_MODEL = 1024
NUM_EXPERTS = 8
LORA_SCALING = 2.0
HIDDEN = 4096
N_TOKENS = 2048
EPAD = 128

HB = 512
NHB = HIDDEN // HB


def _task_gather(task_emb, tids):
    info = plsc.get_sparse_core_info()
    nw = info.num_cores * info.num_subcores
    b_per_w = N_TOKENS // nw
    mesh = plsc.VectorSubcoreMesh(core_axis_name="c", subcore_axis_name="s")

    @functools.partial(
        pl.kernel,
        out_type=jax.ShapeDtypeStruct((N_TOKENS, D_MODEL), jnp.float32),
        mesh=mesh,
        scratch_types=[
            pltpu.VMEM((b_per_w,), jnp.int32),
            pltpu.VMEM((b_per_w, D_MODEL), jnp.float32),
            pltpu.SemaphoreType.DMA,
        ],
    )
    def k(table_hbm, idx_hbm, out_hbm, idx_v, rows_v, sem):
        wid = lax.axis_index("s") * info.num_cores + lax.axis_index("c")
        base = wid * b_per_w
        pltpu.sync_copy(idx_hbm.at[pl.ds(base, b_per_w)], idx_v)
        pltpu.async_copy(table_hbm.at[idx_v], rows_v, sem).wait()
        pltpu.sync_copy(rows_v, out_hbm.at[pl.ds(base, b_per_w)])

    return k(task_emb, tids)


def _gate_body(x_ref, temb_ref, w0_ref, la_ref, lb_ref, xb_ref, cw_ref):
    x_aug = x_ref[...] + temb_ref[...]
    xb_ref[...] = x_aug.astype(jnp.bfloat16)
    delta = lax.dot_general(
        lb_ref[...], la_ref[...], (((0,), (1,)), ((), ())),
        preferred_element_type=jnp.float32)
    w_eff = w0_ref[...] + LORA_SCALING * delta
    logits = lax.dot_general(
        x_aug, w_eff, (((1,), (1,)), ((), ())),
        preferred_element_type=jnp.float32)
    lane = lax.broadcasted_iota(jnp.int32, logits.shape, 1)
    neg = jnp.float32(-1e30)
    logits = jnp.where(lane < NUM_EXPERTS, logits, neg)
    m1 = jnp.max(logits, axis=1, keepdims=True)
    idx1 = jnp.min(jnp.where(logits == m1, lane, NUM_EXPERTS), axis=1,
                   keepdims=True)
    sel1 = lane == idx1
    masked = jnp.where(sel1, neg, logits)
    m2 = jnp.max(masked, axis=1, keepdims=True)
    idx2 = jnp.min(jnp.where(masked == m2, lane, NUM_EXPERTS), axis=1,
                   keepdims=True)
    sel2 = lane == idx2
    b = jnp.exp(m2 - m1)
    w1 = 1.0 / (1.0 + b)
    w2 = b * w1
    cw_ref[...] = jnp.where(sel1, w1, 0.0) + jnp.where(sel2, w2, 0.0)


def _gate(x_flat, temb_rows, base_gate_w, lora_A, lora_B):
    w0_pad = jnp.zeros((EPAD, D_MODEL), jnp.float32).at[:NUM_EXPERTS].set(
        base_gate_w)
    lb_pad = jnp.zeros((lora_B.shape[0], EPAD), jnp.float32).at[
        :, :NUM_EXPERTS].set(lora_B)
    return pl.pallas_call(
        _gate_body,
        out_shape=(
            jax.ShapeDtypeStruct((N_TOKENS, D_MODEL), jnp.bfloat16),
            jax.ShapeDtypeStruct((N_TOKENS, EPAD), jnp.float32),
        ),
    )(x_flat, temb_rows, w0_pad, lora_A, lb_pad)


TB = 512
NTB = N_TOKENS // TB


def _ffn_body(xb_ref, w1_ref, b1_ref, wg_ref, bg_ref, wv_ref, bv_ref, cw_ref,
              out_ref, accg, accv):
    e = pl.program_id(0)
    hb = pl.program_id(1)
    for t in range(NTB):
        tsl = pl.ds(t * TB, TB)
        h = lax.dot_general(xb_ref[tsl, :], w1_ref[0],
                            (((1,), (1,)), ((), ())),
                            preferred_element_type=jnp.float32)
        h = h + b1_ref[0]
        h16 = h.astype(jnp.bfloat16)
        g = lax.dot_general(h16, wg_ref[0], (((1,), (1,)), ((), ())),
                            preferred_element_type=jnp.float32)
        v = lax.dot_general(h16, wv_ref[0], (((1,), (1,)), ((), ())),
                            preferred_element_type=jnp.float32)

        @pl.when(hb == 0)
        def _():
            accg[tsl, :] = g
            accv[tsl, :] = v

        @pl.when(hb != 0)
        def _():
            accg[tsl, :] += g
            accv[tsl, :] += v

        @pl.when(hb == NHB - 1)
        def _():
            gg = accg[tsl, :] + bg_ref[0]
            vv = accv[tsl, :] + bv_ref[0]
            act = gg * (1.0 / (1.0 + jnp.exp(-gg))) * vv
            lane = lax.broadcasted_iota(jnp.int32, (TB, EPAD), 1)
            col = jnp.sum(jnp.where(lane == e, cw_ref[tsl, :], 0.0), axis=1,
                          keepdims=True)
            contrib = act * col

            @pl.when(e == 0)
            def _():
                out_ref[tsl, :] = contrib

            @pl.when(e != 0)
            def _():
                out_ref[tsl, :] += contrib


def _ffn_dense(xb, w1, b1, wg, bg, wv, bv, cw):
    return pl.pallas_call(
        _ffn_body,
        grid=(NUM_EXPERTS, NHB),
        in_specs=[
            pl.BlockSpec((N_TOKENS, D_MODEL), lambda e, h: (0, 0)),
            pl.BlockSpec((1, HB, D_MODEL), lambda e, h: (e, h, 0)),
            pl.BlockSpec((1, 1, HB), lambda e, h: (e, 0, h)),
            pl.BlockSpec((1, D_MODEL, HB), lambda e, h: (e, 0, h)),
            pl.BlockSpec((1, 1, D_MODEL), lambda e, h: (e, 0, 0)),
            pl.BlockSpec((1, D_MODEL, HB), lambda e, h: (e, 0, h)),
            pl.BlockSpec((1, 1, D_MODEL), lambda e, h: (e, 0, 0)),
            pl.BlockSpec((N_TOKENS, EPAD), lambda e, h: (0, 0)),
        ],
        out_specs=pl.BlockSpec((N_TOKENS, D_MODEL), lambda e, h: (0, 0)),
        out_shape=jax.ShapeDtypeStruct((N_TOKENS, D_MODEL), jnp.float32),
        scratch_shapes=[
            pltpu.VMEM((N_TOKENS, D_MODEL), jnp.float32),
            pltpu.VMEM((N_TOKENS, D_MODEL), jnp.float32),
        ],
    )(xb, w1, b1, wg, bg, wv, bv, cw)


def kernel(x, task_emb, base_gate_w, lora_A, lora_B, W1, b1, Wg, bg, Wv, bv,
           task_id_tensor):
    bsz, seqlen, dim = x.shape
    x_flat = x.reshape(-1, dim)
    tids = task_id_tensor.reshape(-1).astype(jnp.int32)
    temb_rows = _task_gather(task_emb, tids)
    xb, cw = _gate(x_flat, temb_rows, base_gate_w, lora_A, lora_B)
    out = _ffn_dense(xb, W1.astype(jnp.bfloat16), b1[:, None, :],
                     Wg.astype(jnp.bfloat16), bg[:, None, :],
                     Wv.astype(jnp.bfloat16), bv[:, None, :], cw)
    return out.reshape(bsz, seqlen, dim)


# speedup vs baseline: 1.1481x; 1.1481x over previous
import functools

import jax
import jax.numpy as jnp
from jax import lax
from jax.experimental import pallas as pl
from jax.experimental.pallas import tpu as pltpu
from jax.experimental.pallas import tpu_sc as plsc

D_MODEL = 1024
NUM_EXPERTS = 8
LORA_SCALING = 2.0
HIDDEN = 4096
N_TOKENS = 2048
N_PAIRS = 2 * N_TOKENS
EPAD = 128

BT = 256
NBLK = (N_PAIRS + NUM_EXPERTS * BT) // BT
PADT = NBLK * BT
HC = 1024
NHC = HIDDEN // HC

NUM_SC_CORES = 2
NUM_SC_SUBCORES = 16
NW = NUM_SC_CORES * NUM_SC_SUBCORES


def _task_gather(task_emb, tids):
    b_per_w = N_TOKENS // NW
    mesh = plsc.VectorSubcoreMesh(core_axis_name="c", subcore_axis_name="s")

    @functools.partial(
        pl.kernel,
        out_type=jax.ShapeDtypeStruct((N_TOKENS, D_MODEL), jnp.float32),
        mesh=mesh,
        scratch_types=[
            pltpu.VMEM((b_per_w,), jnp.int32),
            pltpu.VMEM((b_per_w, D_MODEL), jnp.float32),
            pltpu.SemaphoreType.DMA,
        ],
    )
    def k(table_hbm, idx_hbm, out_hbm, idx_v, rows_v, sem):
        wid = lax.axis_index("s") * NUM_SC_CORES + lax.axis_index("c")
        base = wid * b_per_w
        pltpu.sync_copy(idx_hbm.at[pl.ds(base, b_per_w)], idx_v)
        pltpu.async_copy(table_hbm.at[idx_v], rows_v, sem).wait()
        pltpu.sync_copy(rows_v, out_hbm.at[pl.ds(base, b_per_w)])

    return k(task_emb, tids)


def _gate_body(x_ref, temb_ref, w0_ref, la_ref, lb_ref,
               xb_ref, dest_ref, wp_ref, be_ref, nu_ref):
    x_aug = x_ref[...] + temb_ref[...]
    xb_ref[...] = x_aug.astype(jnp.bfloat16)
    delta = lax.dot_general(
        lb_ref[...], la_ref[...], (((0,), (1,)), ((), ())),
        preferred_element_type=jnp.float32)
    w_eff = w0_ref[...] + LORA_SCALING * delta
    logits = lax.dot_general(
        x_aug, w_eff, (((1,), (1,)), ((), ())),
        preferred_element_type=jnp.float32)
    lane = lax.broadcasted_iota(jnp.int32, logits.shape, 1)
    neg = jnp.float32(-1e30)
    logits = jnp.where(lane < NUM_EXPERTS, logits, neg)
    m1 = jnp.max(logits, axis=1, keepdims=True)
    idx1 = jnp.min(jnp.where(logits == m1, lane, NUM_EXPERTS), axis=1,
                   keepdims=True)
    sel1 = (lane == idx1).astype(jnp.float32)
    masked = jnp.where(lane == idx1, neg, logits)
    m2 = jnp.max(masked, axis=1, keepdims=True)
    idx2 = jnp.min(jnp.where(masked == m2, lane, NUM_EXPERTS), axis=1,
                   keepdims=True)
    sel2 = (lane == idx2).astype(jnp.float32)
    b = jnp.exp(m2 - m1)
    w1 = 1.0 / (1.0 + b)
    w2 = b * w1
    wp_ref[...] = jnp.concatenate([w1, w2], axis=0)

    onehot = jnp.concatenate([sel1, sel2], axis=0)
    incl = onehot
    shift = 1
    while shift < N_PAIRS:
        zeros = jnp.zeros((shift, EPAD), jnp.float32)
        incl = incl + jnp.concatenate(
            [zeros, incl[: N_PAIRS - shift, :]], axis=0)
        shift *= 2
    rank = incl - onehot
    counts = incl[N_PAIRS - 1:N_PAIRS, :]
    padded = jnp.ceil(counts * (1.0 / BT)) * BT
    lrow = lax.broadcasted_iota(jnp.int32, (EPAD, EPAD), 0)
    lcol = lax.broadcasted_iota(jnp.int32, (EPAD, EPAD), 1)
    upper = (lrow < lcol).astype(jnp.float32)
    poff = lax.dot_general(padded, upper, (((1,), (0,)), ((), ())),
                           preferred_element_type=jnp.float32,
                           precision=lax.Precision.HIGHEST)
    dest = jnp.sum(onehot * (rank + poff), axis=1, keepdims=True)
    dest_ref[...] = dest.astype(jnp.int32)
    pend = poff + padded
    ibt = lax.broadcasted_iota(jnp.int32, (NBLK, EPAD), 0).astype(
        jnp.float32) * BT
    ge = jnp.where((ibt >= pend) & (lane[:1, :] < NUM_EXPERTS), 1.0, 0.0)
    be = jnp.sum(ge[:, :], axis=1, keepdims=True)
    be_ref[...] = jnp.minimum(be, NUM_EXPERTS - 1).astype(jnp.int32)
    nused = jnp.sum(jnp.where(lane[:1, :] < NUM_EXPERTS, padded, 0.0),
                    axis=1, keepdims=True) * (1.0 / BT)
    nu_ref[...] = nused.astype(jnp.int32)


def _gate(x_flat, temb_rows, base_gate_w, lora_A, lora_B):
    w0_pad = jnp.zeros((EPAD, D_MODEL), jnp.float32).at[:NUM_EXPERTS].set(
        base_gate_w)
    lb_pad = jnp.zeros((lora_B.shape[0], EPAD), jnp.float32).at[
        :, :NUM_EXPERTS].set(lora_B)
    return pl.pallas_call(
        _gate_body,
        out_shape=(
            jax.ShapeDtypeStruct((N_TOKENS, D_MODEL), jnp.bfloat16),
            jax.ShapeDtypeStruct((N_PAIRS, 1), jnp.int32),
            jax.ShapeDtypeStruct((N_PAIRS, 1), jnp.float32),
            jax.ShapeDtypeStruct((NBLK, 1), jnp.int32),
            jax.ShapeDtypeStruct((1, 1), jnp.int32),
        ),
    )(x_flat, temb_rows, w0_pad, lora_A, lb_pad)


def _sort_scatter(dest, wp):
    mesh = plsc.VectorSubcoreMesh(core_axis_name="c", subcore_axis_name="s")

    @functools.partial(
        pl.kernel,
        out_type=(
            jax.ShapeDtypeStruct((PADT,), jnp.int32),
            jax.ShapeDtypeStruct((PADT,), jnp.float32),
        ),
        mesh=mesh,
        compiler_params=pltpu.CompilerParams(needs_layout_passes=False),
        scratch_types=[
            pltpu.VMEM((N_PAIRS,), jnp.int32),
            pltpu.VMEM((N_PAIRS,), jnp.float32),
            pltpu.VMEM((PADT,), jnp.int32),
            pltpu.VMEM((PADT,), jnp.float32),
        ],
    )
    def k(dest_hbm, wp_hbm, ssrc_hbm, sw_hbm, d_v, w_v, ssrc_v, sw_v):
        wid = lax.axis_index("s") * NUM_SC_CORES + lax.axis_index("c")

        @pl.when(wid == 0)
        def _():
            pltpu.sync_copy(dest_hbm, d_v)
            pltpu.sync_copy(wp_hbm, w_v)
            zi = jnp.zeros((16,), jnp.int32)
            zf = jnp.zeros((16,), jnp.float32)

            def zbody(q, _):
                ssrc_v[pl.ds(q * 16, 16)] = zi
                sw_v[pl.ds(q * 16, 16)] = zf
                return 0

            lax.fori_loop(0, PADT // 16, zbody, 0, unroll=8)

            def sbody(j, _):
                dv = d_v[pl.ds(j * 16, 16)]
                wv = w_v[pl.ds(j * 16, 16)]
                tok = (lax.iota(jnp.int32, 16) + j * 16) & (N_TOKENS - 1)
                plsc.store_scatter(ssrc_v, [dv], tok)
                plsc.store_scatter(sw_v, [dv], wv)
                return 0

            lax.fori_loop(0, N_PAIRS // 16, sbody, 0, unroll=8)
            pltpu.sync_copy(ssrc_v, ssrc_hbm)
            pltpu.sync_copy(sw_v, sw_hbm)

    return k(dest.reshape(N_PAIRS), wp)


def _xgather(ssrc, xb3):
    rows_w = PADT // NW
    half = rows_w // 2
    mesh = plsc.VectorSubcoreMesh(core_axis_name="c", subcore_axis_name="s")

    @functools.partial(
        pl.kernel,
        out_type=jax.ShapeDtypeStruct((PADT, 4, 128), jnp.int32),
        mesh=mesh,
        scratch_types=[
            pltpu.VMEM((half,), jnp.int32),
            pltpu.VMEM((half,), jnp.int32),
            pltpu.VMEM((half, 4, 128), jnp.int32),
            pltpu.SemaphoreType.DMA,
        ],
    )
    def k(ssrc_hbm, xb3_hbm, xs_hbm, idx_a, idx_b, rows_v, sem):
        wid = lax.axis_index("s") * NUM_SC_CORES + lax.axis_index("c")
        base = wid * rows_w
        pltpu.sync_copy(ssrc_hbm.at[pl.ds(base, half)], idx_a)
        pltpu.sync_copy(ssrc_hbm.at[pl.ds(base + half, half)], idx_b)
        pltpu.async_copy(xb3_hbm.at[idx_a], rows_v, sem).wait()
        pltpu.sync_copy(rows_v, xs_hbm.at[pl.ds(base, half)])
        pltpu.async_copy(xb3_hbm.at[idx_b], rows_v, sem).wait()
        pltpu.sync_copy(rows_v, xs_hbm.at[pl.ds(base + half, half)])

    return k(ssrc, xb3)


def _dispatch(dest, wp, xb3):
    ssrc, sw = _sort_scatter(dest, wp)
    return sw, _xgather(ssrc, xb3)


def _ffn_body(be_ref, nu_ref, xs_ref, w1_ref, b1_ref, wg_ref, bg_ref,
              wv_ref, bv_ref, sw_ref, out_ref):
    i = pl.program_id(0)

    @pl.when(i < nu_ref[0])
    def _():
        g = jnp.zeros((BT, D_MODEL), jnp.float32)
        v = jnp.zeros((BT, D_MODEL), jnp.float32)
        for c in range(NHC):
            csl = pl.ds(c * HC, HC)
            h = lax.dot_general(xs_ref[...], w1_ref[0, csl, :],
                                (((1,), (1,)), ((), ())),
                                preferred_element_type=jnp.float32)
            h = h + b1_ref[0, :, csl]
            h16 = h.astype(jnp.bfloat16)
            g = g + lax.dot_general(h16, wg_ref[0, :, csl],
                                    (((1,), (1,)), ((), ())),
                                    preferred_element_type=jnp.float32)
            v = v + lax.dot_general(h16, wv_ref[0, :, csl],
                                    (((1,), (1,)), ((), ())),
                                    preferred_element_type=jnp.float32)
        gg = g + bg_ref[0]
        vv = v + bv_ref[0]
        act = gg * (1.0 / (1.0 + jnp.exp(-gg))) * vv
        wcol = sw_ref[...].astype(jnp.float32)
        out_ref[...] = act * wcol


def _ffn_routed(xs, w1, b1, wg, bg, wv, bv, sw, be, nu):
    grid_spec = pltpu.PrefetchScalarGridSpec(
        num_scalar_prefetch=2,
        grid=(NBLK,),
        in_specs=[
            pl.BlockSpec((BT, D_MODEL), lambda i, be, nu: (i, 0)),
            pl.BlockSpec((1, HIDDEN, D_MODEL), lambda i, be, nu: (be[i], 0, 0)),
            pl.BlockSpec((1, 1, HIDDEN), lambda i, be, nu: (be[i], 0, 0)),
            pl.BlockSpec((1, D_MODEL, HIDDEN), lambda i, be, nu: (be[i], 0, 0)),
            pl.BlockSpec((1, 1, D_MODEL), lambda i, be, nu: (be[i], 0, 0)),
            pl.BlockSpec((1, D_MODEL, HIDDEN), lambda i, be, nu: (be[i], 0, 0)),
            pl.BlockSpec((1, 1, D_MODEL), lambda i, be, nu: (be[i], 0, 0)),
            pl.BlockSpec((BT, 1), lambda i, be, nu: (i, 0)),
        ],
        out_specs=pl.BlockSpec((BT, D_MODEL), lambda i, be, nu: (i, 0)),
    )
    return pl.pallas_call(
        _ffn_body,
        grid_spec=grid_spec,
        out_shape=jax.ShapeDtypeStruct((PADT, D_MODEL), jnp.float32),
    )(be, nu, xs, w1, b1, wg, bg, wv, bv, sw)


def _combine(dest, out_s):
    tok_w = N_TOKENS // NW
    ck = 32
    mesh = plsc.VectorSubcoreMesh(core_axis_name="c", subcore_axis_name="s")

    @functools.partial(
        pl.kernel,
        out_type=jax.ShapeDtypeStruct((N_TOKENS, 8, 128), jnp.float32),
        mesh=mesh,
        scratch_types=[
            pltpu.VMEM((tok_w,), jnp.int32),
            pltpu.VMEM((tok_w,), jnp.int32),
            pltpu.VMEM((ck, 8, 128), jnp.float32),
            pltpu.VMEM((ck, 8, 128), jnp.float32),
            pltpu.SemaphoreType.DMA,
        ],
    )
    def k(dest_hbm, os_hbm, fin_hbm, d0_v, d1_v, r0, r1, sem):
        wid = lax.axis_index("s") * NUM_SC_CORES + lax.axis_index("c")
        base = wid * tok_w
        pltpu.sync_copy(dest_hbm.at[pl.ds(base, tok_w)], d0_v)
        pltpu.sync_copy(dest_hbm.at[pl.ds(N_TOKENS + base, tok_w)], d1_v)
        for c in range(tok_w // ck):
            pltpu.async_copy(os_hbm.at[d0_v.at[pl.ds(c * ck, ck)]], r0,
                             sem).wait()
            pltpu.async_copy(os_hbm.at[d1_v.at[pl.ds(c * ck, ck)]], r1,
                             sem).wait()

            def abody(r, _):
                for s in range(8):
                    for l in range(8):
                        lsl = pl.ds(l * 16, 16)
                        r0[r, s, lsl] = r0[r, s, lsl] + r1[r, s, lsl]
                return 0

            lax.fori_loop(0, ck, abody, 0)
            pltpu.sync_copy(r0, fin_hbm.at[pl.ds(base + c * ck, ck)])

    return k(dest.reshape(N_PAIRS), out_s)


def kernel(x, task_emb, base_gate_w, lora_A, lora_B, W1, b1, Wg, bg, Wv, bv,
           task_id_tensor):
    bsz, seqlen, dim = x.shape
    x_flat = x.reshape(-1, dim)
    tids = task_id_tensor.reshape(-1).astype(jnp.int32)
    temb_rows = _task_gather(task_emb, tids)
    xb, dest, wp, be, nu = _gate(x_flat, temb_rows, base_gate_w, lora_A,
                                 lora_B)
    xpk = lax.bitcast_convert_type(
        xb.reshape(N_TOKENS, D_MODEL // 2, 2), jnp.int32)
    sw, xs3 = _dispatch(dest, wp.reshape(N_PAIRS), xpk.reshape(N_TOKENS, 4,
                                                               128))
    xs = lax.bitcast_convert_type(
        xs3.reshape(PADT, D_MODEL // 2), jnp.bfloat16).reshape(PADT, D_MODEL)
    out_s = _ffn_routed(xs,
                        W1.astype(jnp.bfloat16), b1[:, None, :],
                        Wg.astype(jnp.bfloat16), bg[:, None, :],
                        Wv.astype(jnp.bfloat16), bv[:, None, :],
                        sw.reshape(PADT, 1), be.reshape(NBLK), nu.reshape(1))
    fin = _combine(dest, out_s.reshape(PADT, 8, 128))
    return fin.reshape(bsz, seqlen, dim)


# speedup vs baseline: 1.5173x; 1.3216x over previous
import functools

import jax
import jax.numpy as jnp
from jax import lax
from jax.experimental import pallas as pl
from jax.experimental.pallas import tpu as pltpu
from jax.experimental.pallas import tpu_sc as plsc

D_MODEL = 1024
NUM_EXPERTS = 8
NUM_TASKS = 64
LORA_SCALING = 2.0
HIDDEN = 4096
N_TOKENS = 2048
N_PAIRS = 2 * N_TOKENS
EPAD = 128

BT = 256
NBLK = (N_PAIRS + NUM_EXPERTS * BT) // BT
PADT = NBLK * BT
HC = 1024
NHC = HIDDEN // HC

NUM_SC_CORES = 2
NUM_SC_SUBCORES = 16
NW = NUM_SC_CORES * NUM_SC_SUBCORES


def _gate_body(x_ref, tid_ref, temb_ref, w0_ref, la_ref, lb_ref,
               xa_ref, dest_ref, wp_ref, be_ref, nu_ref):
    t64 = lax.broadcasted_iota(jnp.int32, (N_TOKENS, NUM_TASKS), 1)
    oh_t = (t64 == tid_ref[...]).astype(jnp.float32)
    temb = lax.dot_general(oh_t, temb_ref[...], (((1,), (0,)), ((), ())),
                           preferred_element_type=jnp.float32,
                           precision=lax.Precision.HIGHEST)
    x_aug = x_ref[...] + temb
    xa_ref[...] = x_aug
    delta = lax.dot_general(
        lb_ref[...], la_ref[...], (((0,), (1,)), ((), ())),
        preferred_element_type=jnp.float32)
    w_eff = w0_ref[...] + LORA_SCALING * delta
    logits = lax.dot_general(
        x_aug, w_eff, (((1,), (1,)), ((), ())),
        preferred_element_type=jnp.float32)
    lane = lax.broadcasted_iota(jnp.int32, logits.shape, 1)
    neg = jnp.float32(-1e30)
    logits = jnp.where(lane < NUM_EXPERTS, logits, neg)
    m1 = jnp.max(logits, axis=1, keepdims=True)
    idx1 = jnp.min(jnp.where(logits == m1, lane, NUM_EXPERTS), axis=1,
                   keepdims=True)
    sel1 = (lane == idx1).astype(jnp.float32)
    masked = jnp.where(lane == idx1, neg, logits)
    m2 = jnp.max(masked, axis=1, keepdims=True)
    idx2 = jnp.min(jnp.where(masked == m2, lane, NUM_EXPERTS), axis=1,
                   keepdims=True)
    sel2 = (lane == idx2).astype(jnp.float32)
    b = jnp.exp(m2 - m1)
    w1 = 1.0 / (1.0 + b)
    w2 = b * w1
    wp_ref[...] = jnp.concatenate([w1, w2], axis=0)

    onehot = jnp.concatenate([sel1, sel2], axis=0)
    incl = onehot
    shift = 1
    while shift < N_PAIRS:
        zeros = jnp.zeros((shift, EPAD), jnp.float32)
        incl = incl + jnp.concatenate(
            [zeros, incl[: N_PAIRS - shift, :]], axis=0)
        shift *= 2
    rank = incl - onehot
    counts = incl[N_PAIRS - 1:N_PAIRS, :]
    padded = jnp.ceil(counts * (1.0 / BT)) * BT
    lrow = lax.broadcasted_iota(jnp.int32, (EPAD, EPAD), 0)
    lcol = lax.broadcasted_iota(jnp.int32, (EPAD, EPAD), 1)
    upper = (lrow < lcol).astype(jnp.float32)
    poff = lax.dot_general(padded, upper, (((1,), (0,)), ((), ())),
                           preferred_element_type=jnp.float32,
                           precision=lax.Precision.HIGHEST)
    dest = jnp.sum(onehot * (rank + poff), axis=1, keepdims=True)
    dest_ref[...] = dest.astype(jnp.int32)
    pend = poff + padded
    ibt = lax.broadcasted_iota(jnp.int32, (NBLK, EPAD), 0).astype(
        jnp.float32) * BT
    ge = jnp.where((ibt >= pend) & (lane[:1, :] < NUM_EXPERTS), 1.0, 0.0)
    be = jnp.sum(ge[:, :], axis=1, keepdims=True)
    be_ref[...] = jnp.minimum(be, NUM_EXPERTS - 1).astype(jnp.int32)
    nused = jnp.sum(jnp.where(lane[:1, :] < NUM_EXPERTS, padded, 0.0),
                    axis=1, keepdims=True) * (1.0 / BT)
    nu_ref[...] = nused.astype(jnp.int32)


def _gate(x_flat, tids2, task_emb, base_gate_w, lora_A, lora_B):
    w0_pad = jnp.zeros((EPAD, D_MODEL), jnp.float32).at[:NUM_EXPERTS].set(
        base_gate_w)
    lb_pad = jnp.zeros((lora_B.shape[0], EPAD), jnp.float32).at[
        :, :NUM_EXPERTS].set(lora_B)
    return pl.pallas_call(
        _gate_body,
        out_shape=(
            jax.ShapeDtypeStruct((N_TOKENS, D_MODEL), jnp.float32),
            jax.ShapeDtypeStruct((N_PAIRS, 1), jnp.int32),
            jax.ShapeDtypeStruct((N_PAIRS, 1), jnp.float32),
            jax.ShapeDtypeStruct((NBLK, 1), jnp.int32),
            jax.ShapeDtypeStruct((1, 1), jnp.int32),
        ),
        name="moe_gate",
    )(x_flat, tids2, task_emb, w0_pad, lora_A, lb_pad)


def _sort_scatter(dest, wp):
    mesh = plsc.VectorSubcoreMesh(core_axis_name="c", subcore_axis_name="s")

    @functools.partial(
        pl.kernel,
        out_type=(
            jax.ShapeDtypeStruct((PADT,), jnp.int32),
            jax.ShapeDtypeStruct((PADT,), jnp.float32),
        ),
        mesh=mesh,
        compiler_params=pltpu.CompilerParams(needs_layout_passes=False),
        scratch_types=[
            pltpu.VMEM((N_PAIRS,), jnp.int32),
            pltpu.VMEM((N_PAIRS,), jnp.float32),
            pltpu.VMEM((PADT,), jnp.int32),
            pltpu.VMEM((PADT,), jnp.float32),
        ],
        name="moe_sort",
    )
    def k(dest_hbm, wp_hbm, ssrc_hbm, sw_hbm, d_v, w_v, ssrc_v, sw_v):
        wid = lax.axis_index("s") * NUM_SC_CORES + lax.axis_index("c")

        @pl.when(wid == 0)
        def _():
            pltpu.sync_copy(dest_hbm, d_v)
            pltpu.sync_copy(wp_hbm, w_v)
            zi = jnp.zeros((16,), jnp.int32)
            zf = jnp.zeros((16,), jnp.float32)

            def zbody(q, _):
                ssrc_v[pl.ds(q * 16, 16)] = zi
                sw_v[pl.ds(q * 16, 16)] = zf
                return 0

            lax.fori_loop(0, PADT // 16, zbody, 0, unroll=8)

            def sbody(j, _):
                dv = d_v[pl.ds(j * 16, 16)]
                wv = w_v[pl.ds(j * 16, 16)]
                tok = (lax.iota(jnp.int32, 16) + j * 16) & (N_TOKENS - 1)
                plsc.store_scatter(ssrc_v, [dv], tok)
                plsc.store_scatter(sw_v, [dv], wv)
                return 0

            lax.fori_loop(0, N_PAIRS // 16, sbody, 0, unroll=8)
            pltpu.sync_copy(ssrc_v, ssrc_hbm)
            pltpu.sync_copy(sw_v, sw_hbm)

    return k(dest.reshape(N_PAIRS), wp)


def _xgather(ssrc, xa):
    rows_w = PADT // NW
    half = rows_w // 2
    mesh = plsc.VectorSubcoreMesh(core_axis_name="c", subcore_axis_name="s")

    @functools.partial(
        pl.kernel,
        out_type=jax.ShapeDtypeStruct((PADT, D_MODEL), jnp.float32),
        mesh=mesh,
        scratch_types=[
            pltpu.VMEM((half,), jnp.int32),
            pltpu.VMEM((half,), jnp.int32),
            pltpu.VMEM((half, D_MODEL), jnp.float32),
            pltpu.SemaphoreType.DMA,
        ],
        name="moe_xgather",
    )
    def k(ssrc_hbm, xa_hbm, xs_hbm, idx_a, idx_b, rows_v, sem):
        wid = lax.axis_index("s") * NUM_SC_CORES + lax.axis_index("c")
        base = wid * rows_w
        pltpu.sync_copy(ssrc_hbm.at[pl.ds(base, half)], idx_a)
        pltpu.sync_copy(ssrc_hbm.at[pl.ds(base + half, half)], idx_b)
        pltpu.async_copy(xa_hbm.at[idx_a], rows_v, sem).wait()
        pltpu.sync_copy(rows_v, xs_hbm.at[pl.ds(base, half)])
        pltpu.async_copy(xa_hbm.at[idx_b], rows_v, sem).wait()
        pltpu.sync_copy(rows_v, xs_hbm.at[pl.ds(base + half, half)])

    return k(ssrc, xa)


def _ffn_body(be_ref, nu_ref, xs_ref, w1_ref, b1_ref, wg_ref, bg_ref,
              wv_ref, bv_ref, sw_ref, out_ref):
    i = pl.program_id(0)

    @pl.when(i < nu_ref[0])
    def _():
        xs16 = xs_ref[...].astype(jnp.bfloat16)
        g = jnp.zeros((BT, D_MODEL), jnp.float32)
        v = jnp.zeros((BT, D_MODEL), jnp.float32)
        for c in range(NHC):
            csl = pl.ds(c * HC, HC)
            h = lax.dot_general(xs16, w1_ref[0, csl, :],
                                (((1,), (1,)), ((), ())),
                                preferred_element_type=jnp.float32)
            h = h + b1_ref[0, :, csl]
            h16 = h.astype(jnp.bfloat16)
            g = g + lax.dot_general(h16, wg_ref[0, :, csl],
                                    (((1,), (1,)), ((), ())),
                                    preferred_element_type=jnp.float32)
            v = v + lax.dot_general(h16, wv_ref[0, :, csl],
                                    (((1,), (1,)), ((), ())),
                                    preferred_element_type=jnp.float32)
        gg = g + bg_ref[0]
        vv = v + bv_ref[0]
        act = gg * (1.0 / (1.0 + jnp.exp(-gg))) * vv
        wcol = sw_ref[...].astype(jnp.float32)
        out_ref[...] = act * wcol


def _ffn_routed(xs, w1, b1, wg, bg, wv, bv, sw, be, nu):
    grid_spec = pltpu.PrefetchScalarGridSpec(
        num_scalar_prefetch=2,
        grid=(NBLK,),
        in_specs=[
            pl.BlockSpec((BT, D_MODEL), lambda i, be, nu: (i, 0)),
            pl.BlockSpec((1, HIDDEN, D_MODEL), lambda i, be, nu: (be[i], 0, 0)),
            pl.BlockSpec((1, 1, HIDDEN), lambda i, be, nu: (be[i], 0, 0)),
            pl.BlockSpec((1, D_MODEL, HIDDEN), lambda i, be, nu: (be[i], 0, 0)),
            pl.BlockSpec((1, 1, D_MODEL), lambda i, be, nu: (be[i], 0, 0)),
            pl.BlockSpec((1, D_MODEL, HIDDEN), lambda i, be, nu: (be[i], 0, 0)),
            pl.BlockSpec((1, 1, D_MODEL), lambda i, be, nu: (be[i], 0, 0)),
            pl.BlockSpec((BT, 1), lambda i, be, nu: (i, 0)),
        ],
        out_specs=pl.BlockSpec((BT, D_MODEL), lambda i, be, nu: (i, 0)),
    )
    return pl.pallas_call(
        _ffn_body,
        grid_spec=grid_spec,
        out_shape=jax.ShapeDtypeStruct((PADT, D_MODEL), jnp.float32),
        name="moe_ffn",
    )(be, nu, xs, w1, b1, wg, bg, wv, bv, sw)


def _combine(dest, out_s):
    tok_w = N_TOKENS // NW
    ck = 32
    mesh = plsc.VectorSubcoreMesh(core_axis_name="c", subcore_axis_name="s")

    @functools.partial(
        pl.kernel,
        out_type=jax.ShapeDtypeStruct((N_TOKENS, D_MODEL), jnp.float32),
        mesh=mesh,
        compiler_params=pltpu.CompilerParams(needs_layout_passes=False),
        scratch_types=[
            pltpu.VMEM((tok_w,), jnp.int32),
            pltpu.VMEM((tok_w,), jnp.int32),
            pltpu.VMEM((ck, D_MODEL), jnp.float32),
            pltpu.VMEM((ck, D_MODEL), jnp.float32),
            pltpu.SemaphoreType.DMA,
        ],
        name="moe_combine",
    )
    def k(dest_hbm, os_hbm, fin_hbm, d0_v, d1_v, r0, r1, sem):
        wid = lax.axis_index("s") * NUM_SC_CORES + lax.axis_index("c")
        base = wid * tok_w
        pltpu.sync_copy(dest_hbm.at[pl.ds(base, tok_w)], d0_v)
        pltpu.sync_copy(dest_hbm.at[pl.ds(N_TOKENS + base, tok_w)], d1_v)
        for c in range(tok_w // ck):
            pltpu.async_copy(os_hbm.at[d0_v.at[pl.ds(c * ck, ck)]], r0,
                             sem).wait()
            pltpu.async_copy(os_hbm.at[d1_v.at[pl.ds(c * ck, ck)]], r1,
                             sem).wait()

            def abody(r, _):
                for l in range(D_MODEL // 16):
                    lsl = pl.ds(l * 16, 16)
                    r0[r, lsl] = r0[r, lsl] + r1[r, lsl]
                return 0

            lax.fori_loop(0, ck, abody, 0)
            pltpu.sync_copy(r0, fin_hbm.at[pl.ds(base + c * ck, ck)])

    return k(dest.reshape(N_PAIRS), out_s)


def kernel(x, task_emb, base_gate_w, lora_A, lora_B, W1, b1, Wg, bg, Wv, bv,
           task_id_tensor):
    bsz, seqlen, dim = x.shape
    x_flat = x.reshape(-1, dim)
    tids2 = task_id_tensor.reshape(-1, 1).astype(jnp.int32)
    xa, dest, wp, be, nu = _gate(x_flat, tids2, task_emb, base_gate_w,
                                 lora_A, lora_B)
    ssrc, sw = _sort_scatter(dest, wp.reshape(N_PAIRS))
    xs = _xgather(ssrc, xa)
    out_s = _ffn_routed(xs, W1.astype(jnp.bfloat16), b1[:, None, :],
                        Wg.astype(jnp.bfloat16), bg[:, None, :],
                        Wv.astype(jnp.bfloat16), bv[:, None, :],
                        sw.reshape(PADT, 1), be.reshape(NBLK), nu.reshape(1))
    fin = _combine(dest, out_s)
    return fin.reshape(bsz, seqlen, dim)


# speedup vs baseline: 1.7835x; 1.1755x over previous
import functools

import jax
import jax.numpy as jnp
from jax import lax
from jax.experimental import pallas as pl
from jax.experimental.pallas import tpu as pltpu
from jax.experimental.pallas import tpu_sc as plsc

D_MODEL = 1024
NUM_EXPERTS = 8
NUM_TASKS = 64
LORA_SCALING = 2.0
HIDDEN = 4096
N_TOKENS = 2048
N_PAIRS = 2 * N_TOKENS
EPAD = 128

BT = 256
NBLK = (N_PAIRS + NUM_EXPERTS * BT) // BT
PADT = NBLK * BT
HC = 1024
NHC = HIDDEN // HC

NUM_SC_CORES = 2
NUM_SC_SUBCORES = 16
NW = NUM_SC_CORES * NUM_SC_SUBCORES


def _gate_body(x_ref, tid_ref, temb_ref, w0_ref, la_ref, lb_ref,
               xa_ref, dest_ref, wp_ref, be_ref, nu_ref):
    t64 = lax.broadcasted_iota(jnp.int32, (N_TOKENS, NUM_TASKS), 1)
    oh_t = (t64 == tid_ref[...]).astype(jnp.float32)
    temb = lax.dot_general(oh_t, temb_ref[...], (((1,), (0,)), ((), ())),
                           preferred_element_type=jnp.float32,
                           precision=lax.Precision.HIGHEST)
    x_aug = x_ref[...] + temb
    xa_ref[...] = x_aug.astype(jnp.bfloat16)
    delta = lax.dot_general(
        lb_ref[...], la_ref[...], (((0,), (1,)), ((), ())),
        preferred_element_type=jnp.float32)
    w_eff = w0_ref[...] + LORA_SCALING * delta
    logits = lax.dot_general(
        x_aug, w_eff, (((1,), (1,)), ((), ())),
        preferred_element_type=jnp.float32)
    lane = lax.broadcasted_iota(jnp.int32, logits.shape, 1)
    neg = jnp.float32(-1e30)
    logits = jnp.where(lane < NUM_EXPERTS, logits, neg)
    m1 = jnp.max(logits, axis=1, keepdims=True)
    idx1 = jnp.min(jnp.where(logits == m1, lane, NUM_EXPERTS), axis=1,
                   keepdims=True)
    sel1 = (lane == idx1).astype(jnp.float32)
    masked = jnp.where(lane == idx1, neg, logits)
    m2 = jnp.max(masked, axis=1, keepdims=True)
    idx2 = jnp.min(jnp.where(masked == m2, lane, NUM_EXPERTS), axis=1,
                   keepdims=True)
    sel2 = (lane == idx2).astype(jnp.float32)
    b = jnp.exp(m2 - m1)
    w1 = 1.0 / (1.0 + b)
    w2 = b * w1
    wp_ref[...] = jnp.concatenate([w1, w2], axis=0)

    onehot = jnp.concatenate([sel1, sel2], axis=0)
    incl = onehot
    shift = 1
    while shift < N_PAIRS:
        zeros = jnp.zeros((shift, EPAD), jnp.float32)
        incl = incl + jnp.concatenate(
            [zeros, incl[: N_PAIRS - shift, :]], axis=0)
        shift *= 2
    rank = incl - onehot
    counts = incl[N_PAIRS - 1:N_PAIRS, :]
    padded = jnp.ceil(counts * (1.0 / BT)) * BT
    lrow = lax.broadcasted_iota(jnp.int32, (EPAD, EPAD), 0)
    lcol = lax.broadcasted_iota(jnp.int32, (EPAD, EPAD), 1)
    upper = (lrow < lcol).astype(jnp.float32)
    poff = lax.dot_general(padded, upper, (((1,), (0,)), ((), ())),
                           preferred_element_type=jnp.float32,
                           precision=lax.Precision.HIGHEST)
    dest = jnp.sum(onehot * (rank + poff), axis=1, keepdims=True)
    dest_ref[...] = dest.astype(jnp.int32)
    pend = poff + padded
    ibt = lax.broadcasted_iota(jnp.int32, (NBLK, EPAD), 0).astype(
        jnp.float32) * BT
    ge = jnp.where((ibt >= pend) & (lane[:1, :] < NUM_EXPERTS), 1.0, 0.0)
    be = jnp.sum(ge[:, :], axis=1, keepdims=True)
    be_ref[...] = jnp.minimum(be, NUM_EXPERTS - 1).astype(jnp.int32)
    nused = jnp.sum(jnp.where(lane[:1, :] < NUM_EXPERTS, padded, 0.0),
                    axis=1, keepdims=True) * (1.0 / BT)
    nu_ref[...] = nused.astype(jnp.int32)


def _gate(x_flat, tids2, task_emb, base_gate_w, lora_A, lora_B):
    w0_pad = jnp.zeros((EPAD, D_MODEL), jnp.float32).at[:NUM_EXPERTS].set(
        base_gate_w)
    lb_pad = jnp.zeros((lora_B.shape[0], EPAD), jnp.float32).at[
        :, :NUM_EXPERTS].set(lora_B)
    return pl.pallas_call(
        _gate_body,
        out_shape=(
            jax.ShapeDtypeStruct((N_TOKENS, D_MODEL), jnp.bfloat16),
            jax.ShapeDtypeStruct((N_PAIRS, 1), jnp.int32),
            jax.ShapeDtypeStruct((N_PAIRS, 1), jnp.float32),
            jax.ShapeDtypeStruct((NBLK, 1), jnp.int32),
            jax.ShapeDtypeStruct((1, 1), jnp.int32),
        ),
        name="moe_gate",
    )(x_flat, tids2, task_emb, w0_pad, lora_A, lb_pad)


def _sort_scatter(dest, wp):
    mesh = plsc.VectorSubcoreMesh(core_axis_name="c", subcore_axis_name="s")

    @functools.partial(
        pl.kernel,
        out_type=(
            jax.ShapeDtypeStruct((PADT,), jnp.int32),
            jax.ShapeDtypeStruct((PADT,), jnp.float32),
        ),
        mesh=mesh,
        compiler_params=pltpu.CompilerParams(needs_layout_passes=False),
        scratch_types=[
            pltpu.VMEM((N_PAIRS,), jnp.int32),
            pltpu.VMEM((N_PAIRS,), jnp.float32),
            pltpu.VMEM((PADT,), jnp.int32),
            pltpu.VMEM((PADT,), jnp.float32),
        ],
        name="moe_sort",
    )
    def k(dest_hbm, wp_hbm, ssrc_hbm, sw_hbm, d_v, w_v, ssrc_v, sw_v):
        wid = lax.axis_index("s") * NUM_SC_CORES + lax.axis_index("c")

        @pl.when(wid == 0)
        def _():
            pltpu.sync_copy(dest_hbm, d_v)
            pltpu.sync_copy(wp_hbm, w_v)
            zi = jnp.zeros((16,), jnp.int32)
            zf = jnp.zeros((16,), jnp.float32)

            def zbody(q, _):
                ssrc_v[pl.ds(q * 16, 16)] = zi
                sw_v[pl.ds(q * 16, 16)] = zf
                return 0

            lax.fori_loop(0, PADT // 16, zbody, 0, unroll=8)

            def sbody(j, _):
                dv = d_v[pl.ds(j * 16, 16)]
                wv = w_v[pl.ds(j * 16, 16)]
                tok = (lax.iota(jnp.int32, 16) + j * 16) & (N_TOKENS - 1)
                plsc.store_scatter(ssrc_v, [dv], tok)
                plsc.store_scatter(sw_v, [dv], wv)
                return 0

            lax.fori_loop(0, N_PAIRS // 16, sbody, 0, unroll=8)
            pltpu.sync_copy(ssrc_v, ssrc_hbm)
            pltpu.sync_copy(sw_v, sw_hbm)

    return k(dest.reshape(N_PAIRS), wp)


def _ffn_up_body(be_ref, nu_ref, ssrc_ref, xb_ref, w1_ref, b1_ref, h_ref):
    i = pl.program_id(0)

    @pl.when(i < nu_ref[0])
    def _():
        tcol = lax.broadcasted_iota(jnp.int32, (BT, N_TOKENS), 1)
        p = (tcol == ssrc_ref[...]).astype(jnp.bfloat16)
        xs = lax.dot_general(p, xb_ref[...], (((1,), (0,)), ((), ())),
                             preferred_element_type=jnp.float32)
        xs16 = xs.astype(jnp.bfloat16)
        h = lax.dot_general(xs16, w1_ref[0], (((1,), (1,)), ((), ())),
                            preferred_element_type=jnp.float32)
        h = h + b1_ref[0]
        h_ref[...] = h.astype(jnp.bfloat16)


def _ffn_up(ssrc2, xb, w1, b1, be, nu):
    grid_spec = pltpu.PrefetchScalarGridSpec(
        num_scalar_prefetch=2,
        grid=(NBLK,),
        in_specs=[
            pl.BlockSpec((BT, 1), lambda i, be, nu: (i, 0)),
            pl.BlockSpec((N_TOKENS, D_MODEL), lambda i, be, nu: (0, 0)),
            pl.BlockSpec((1, HIDDEN, D_MODEL), lambda i, be, nu: (be[i], 0, 0)),
            pl.BlockSpec((1, 1, HIDDEN), lambda i, be, nu: (be[i], 0, 0)),
        ],
        out_specs=pl.BlockSpec((BT, HIDDEN), lambda i, be, nu: (i, 0)),
    )
    return pl.pallas_call(
        _ffn_up_body,
        grid_spec=grid_spec,
        out_shape=jax.ShapeDtypeStruct((PADT, HIDDEN), jnp.bfloat16),
        name="moe_ffn_up",
    )(be, nu, ssrc2, xb, w1, b1)


def _ffn_down_body(be_ref, nu_ref, h_ref, wg_ref, bg_ref, wv_ref, bv_ref,
                   sw_ref, out_ref):
    i = pl.program_id(0)

    @pl.when(i < nu_ref[0])
    def _():
        h16 = h_ref[...]
        g = lax.dot_general(h16, wg_ref[0], (((1,), (1,)), ((), ())),
                            preferred_element_type=jnp.float32)
        v = lax.dot_general(h16, wv_ref[0], (((1,), (1,)), ((), ())),
                            preferred_element_type=jnp.float32)
        gg = g + bg_ref[0]
        vv = v + bv_ref[0]
        act = gg * (1.0 / (1.0 + jnp.exp(-gg))) * vv
        wcol = sw_ref[...].astype(jnp.float32)
        out_ref[...] = act * wcol


def _ffn_down(h_s, wg, bg, wv, bv, sw, be, nu):
    grid_spec = pltpu.PrefetchScalarGridSpec(
        num_scalar_prefetch=2,
        grid=(NBLK,),
        in_specs=[
            pl.BlockSpec((BT, HIDDEN), lambda i, be, nu: (i, 0)),
            pl.BlockSpec((1, D_MODEL, HIDDEN), lambda i, be, nu: (be[i], 0, 0)),
            pl.BlockSpec((1, 1, D_MODEL), lambda i, be, nu: (be[i], 0, 0)),
            pl.BlockSpec((1, D_MODEL, HIDDEN), lambda i, be, nu: (be[i], 0, 0)),
            pl.BlockSpec((1, 1, D_MODEL), lambda i, be, nu: (be[i], 0, 0)),
            pl.BlockSpec((BT, 1), lambda i, be, nu: (i, 0)),
        ],
        out_specs=pl.BlockSpec((BT, D_MODEL), lambda i, be, nu: (i, 0)),
    )
    return pl.pallas_call(
        _ffn_down_body,
        grid_spec=grid_spec,
        out_shape=jax.ShapeDtypeStruct((PADT, D_MODEL), jnp.float32),
        name="moe_ffn_down",
    )(be, nu, h_s, wg, bg, wv, bv, sw)


def _combine(dest, out_s):
    tok_w = N_TOKENS // NW
    ck = 32
    mesh = plsc.VectorSubcoreMesh(core_axis_name="c", subcore_axis_name="s")

    @functools.partial(
        pl.kernel,
        out_type=jax.ShapeDtypeStruct((N_TOKENS, D_MODEL), jnp.float32),
        mesh=mesh,
        compiler_params=pltpu.CompilerParams(needs_layout_passes=False),
        scratch_types=[
            pltpu.VMEM((tok_w,), jnp.int32),
            pltpu.VMEM((tok_w,), jnp.int32),
            pltpu.VMEM((ck, D_MODEL), jnp.float32),
            pltpu.VMEM((ck, D_MODEL), jnp.float32),
            pltpu.SemaphoreType.DMA,
        ],
        name="moe_combine",
    )
    def k(dest_hbm, os_hbm, fin_hbm, d0_v, d1_v, r0, r1, sem):
        wid = lax.axis_index("s") * NUM_SC_CORES + lax.axis_index("c")
        base = wid * tok_w
        pltpu.sync_copy(dest_hbm.at[pl.ds(base, tok_w)], d0_v)
        pltpu.sync_copy(dest_hbm.at[pl.ds(N_TOKENS + base, tok_w)], d1_v)
        for c in range(tok_w // ck):
            pltpu.async_copy(os_hbm.at[d0_v.at[pl.ds(c * ck, ck)]], r0,
                             sem).wait()
            pltpu.async_copy(os_hbm.at[d1_v.at[pl.ds(c * ck, ck)]], r1,
                             sem).wait()

            def abody(r, _):
                for l in range(D_MODEL // 16):
                    lsl = pl.ds(l * 16, 16)
                    r0[r, lsl] = r0[r, lsl] + r1[r, lsl]
                return 0

            lax.fori_loop(0, ck, abody, 0)
            pltpu.sync_copy(r0, fin_hbm.at[pl.ds(base + c * ck, ck)])

    return k(dest.reshape(N_PAIRS), out_s)


def kernel(x, task_emb, base_gate_w, lora_A, lora_B, W1, b1, Wg, bg, Wv, bv,
           task_id_tensor):
    bsz, seqlen, dim = x.shape
    x_flat = x.reshape(-1, dim)
    tids2 = task_id_tensor.reshape(-1, 1).astype(jnp.int32)
    xb, dest, wp, be, nu = _gate(x_flat, tids2, task_emb, base_gate_w,
                                 lora_A, lora_B)
    ssrc, sw = _sort_scatter(dest, wp.reshape(N_PAIRS))
    ben, nun = be.reshape(NBLK), nu.reshape(1)
    h_s = _ffn_up(ssrc.reshape(PADT, 1), xb, W1.astype(jnp.bfloat16),
                  b1[:, None, :].astype(jnp.bfloat16), ben, nun)
    out_s = _ffn_down(h_s, Wg.astype(jnp.bfloat16), bg[:, None, :],
                      Wv.astype(jnp.bfloat16), bv[:, None, :],
                      sw.reshape(PADT, 1), ben, nun)
    fin = _combine(dest, out_s)
    return fin.reshape(bsz, seqlen, dim)


# speedup vs baseline: 2.1941x; 1.2302x over previous
import functools

import jax
import jax.numpy as jnp
from jax import lax
from jax.experimental import pallas as pl
from jax.experimental.pallas import tpu as pltpu
from jax.experimental.pallas import tpu_sc as plsc

D_MODEL = 1024
NUM_EXPERTS = 8
NUM_TASKS = 64
LORA_SCALING = 2.0
HIDDEN = 4096
N_TOKENS = 2048
N_PAIRS = 2 * N_TOKENS
EPAD = 128

BT = 256
NBLK = (N_PAIRS + NUM_EXPERTS * BT) // BT
PADT = NBLK * BT
HC = 1024
NHC = HIDDEN // HC

NUM_SC_CORES = 2
NUM_SC_SUBCORES = 16
NW = NUM_SC_CORES * NUM_SC_SUBCORES


def _gate_body(x_ref, tid_ref, temb_ref, w0_ref, la_ref, lb_ref,
               xa_ref, dest_ref, wp_ref, be_ref, nu_ref):
    t64 = lax.broadcasted_iota(jnp.int32, (N_TOKENS, NUM_TASKS), 1)
    oh_t = (t64 == tid_ref[...]).astype(jnp.float32)
    temb = lax.dot_general(oh_t, temb_ref[...], (((1,), (0,)), ((), ())),
                           preferred_element_type=jnp.float32,
                           precision=lax.Precision.HIGHEST)
    x_aug = x_ref[...] + temb
    xa_ref[...] = x_aug.astype(jnp.bfloat16)
    delta = lax.dot_general(
        lb_ref[...], la_ref[...], (((0,), (1,)), ((), ())),
        preferred_element_type=jnp.float32)
    w_eff = w0_ref[...] + LORA_SCALING * delta
    logits = lax.dot_general(
        x_aug, w_eff, (((1,), (1,)), ((), ())),
        preferred_element_type=jnp.float32)
    lane = lax.broadcasted_iota(jnp.int32, logits.shape, 1)
    neg = jnp.float32(-1e30)
    logits = jnp.where(lane < NUM_EXPERTS, logits, neg)
    m1 = jnp.max(logits, axis=1, keepdims=True)
    idx1 = jnp.min(jnp.where(logits == m1, lane, NUM_EXPERTS), axis=1,
                   keepdims=True)
    sel1 = (lane == idx1).astype(jnp.float32)
    masked = jnp.where(lane == idx1, neg, logits)
    m2 = jnp.max(masked, axis=1, keepdims=True)
    idx2 = jnp.min(jnp.where(masked == m2, lane, NUM_EXPERTS), axis=1,
                   keepdims=True)
    sel2 = (lane == idx2).astype(jnp.float32)
    b = jnp.exp(m2 - m1)
    w1 = 1.0 / (1.0 + b)
    w2 = b * w1
    wp_ref[...] = jnp.concatenate([w1, w2], axis=0)

    onehot = jnp.concatenate([sel1, sel2], axis=0)
    incl = onehot
    shift = 1
    while shift < N_PAIRS:
        zeros = jnp.zeros((shift, EPAD), jnp.float32)
        incl = incl + jnp.concatenate(
            [zeros, incl[: N_PAIRS - shift, :]], axis=0)
        shift *= 2
    rank = incl - onehot
    counts = incl[N_PAIRS - 1:N_PAIRS, :]
    padded = jnp.ceil(counts * (1.0 / BT)) * BT
    lrow = lax.broadcasted_iota(jnp.int32, (EPAD, EPAD), 0)
    lcol = lax.broadcasted_iota(jnp.int32, (EPAD, EPAD), 1)
    upper = (lrow < lcol).astype(jnp.float32)
    poff = lax.dot_general(padded, upper, (((1,), (0,)), ((), ())),
                           preferred_element_type=jnp.float32,
                           precision=lax.Precision.HIGHEST)
    dest = jnp.sum(onehot * (rank + poff), axis=1, keepdims=True)
    dest_ref[...] = dest.astype(jnp.int32)
    pend = poff + padded
    ibt = lax.broadcasted_iota(jnp.int32, (NBLK, EPAD), 0).astype(
        jnp.float32) * BT
    ge = jnp.where((ibt >= pend) & (lane[:1, :] < NUM_EXPERTS), 1.0, 0.0)
    be = jnp.sum(ge[:, :], axis=1, keepdims=True)
    be_ref[...] = jnp.minimum(be, NUM_EXPERTS - 1).astype(jnp.int32)
    nused = jnp.sum(jnp.where(lane[:1, :] < NUM_EXPERTS, padded, 0.0),
                    axis=1, keepdims=True) * (1.0 / BT)
    nu_ref[...] = nused.astype(jnp.int32)


def _gate(x_flat, tids2, task_emb, base_gate_w, lora_A, lora_B):
    w0_pad = jnp.zeros((EPAD, D_MODEL), jnp.float32).at[:NUM_EXPERTS].set(
        base_gate_w)
    lb_pad = jnp.zeros((lora_B.shape[0], EPAD), jnp.float32).at[
        :, :NUM_EXPERTS].set(lora_B)
    return pl.pallas_call(
        _gate_body,
        out_shape=(
            jax.ShapeDtypeStruct((N_TOKENS, D_MODEL), jnp.bfloat16),
            jax.ShapeDtypeStruct((N_PAIRS, 1), jnp.int32),
            jax.ShapeDtypeStruct((N_PAIRS, 1), jnp.float32),
            jax.ShapeDtypeStruct((NBLK, 1), jnp.int32),
            jax.ShapeDtypeStruct((1, 1), jnp.int32),
        ),
        name="moe_gate",
    )(x_flat, tids2, task_emb, w0_pad, lora_A, lb_pad)


def _sort_scatter(dest, wp):
    mesh = plsc.VectorSubcoreMesh(core_axis_name="c", subcore_axis_name="s")

    @functools.partial(
        pl.kernel,
        out_type=(
            jax.ShapeDtypeStruct((PADT,), jnp.int32),
            jax.ShapeDtypeStruct((PADT,), jnp.float32),
        ),
        mesh=mesh,
        compiler_params=pltpu.CompilerParams(needs_layout_passes=False),
        scratch_types=[
            pltpu.VMEM((N_PAIRS,), jnp.int32),
            pltpu.VMEM((N_PAIRS,), jnp.float32),
            pltpu.VMEM((PADT,), jnp.int32),
            pltpu.VMEM((PADT,), jnp.float32),
        ],
        name="moe_sort",
    )
    def k(dest_hbm, wp_hbm, ssrc_hbm, sw_hbm, d_v, w_v, ssrc_v, sw_v):
        wid = lax.axis_index("s") * NUM_SC_CORES + lax.axis_index("c")

        @pl.when(wid == 0)
        def _():
            pltpu.sync_copy(dest_hbm, d_v)
            pltpu.sync_copy(wp_hbm, w_v)
            zi = jnp.zeros((16,), jnp.int32)
            zf = jnp.zeros((16,), jnp.float32)

            def zbody(q, _):
                ssrc_v[pl.ds(q * 16, 16)] = zi
                sw_v[pl.ds(q * 16, 16)] = zf
                return 0

            lax.fori_loop(0, PADT // 16, zbody, 0, unroll=8)

            def sbody(j, _):
                dv = d_v[pl.ds(j * 16, 16)]
                wv = w_v[pl.ds(j * 16, 16)]
                tok = (lax.iota(jnp.int32, 16) + j * 16) & (N_TOKENS - 1)
                plsc.store_scatter(ssrc_v, [dv], tok)
                plsc.store_scatter(sw_v, [dv], wv)
                return 0

            lax.fori_loop(0, N_PAIRS // 16, sbody, 0, unroll=8)
            pltpu.sync_copy(ssrc_v, ssrc_hbm)
            pltpu.sync_copy(sw_v, sw_hbm)

    return k(dest.reshape(N_PAIRS), wp)


def _ffn_up_body(be_ref, nu_ref, ssrc_ref, xb_ref, w1_ref, b1_ref, h_ref):
    i = pl.program_id(0)

    @pl.when(i < nu_ref[0])
    def _():
        tcol = lax.broadcasted_iota(jnp.int32, (BT, N_TOKENS), 1)
        p = (tcol == ssrc_ref[...]).astype(jnp.bfloat16)
        xs = lax.dot_general(p, xb_ref[...], (((1,), (0,)), ((), ())),
                             preferred_element_type=jnp.float32)
        h = lax.dot_general(xs, w1_ref[0], (((1,), (1,)), ((), ())),
                            preferred_element_type=jnp.float32)
        h = h + b1_ref[0]
        h_ref[...] = h.astype(jnp.bfloat16)


def _ffn_up(ssrc2, xb, w1, b1, be, nu):
    grid_spec = pltpu.PrefetchScalarGridSpec(
        num_scalar_prefetch=2,
        grid=(NBLK,),
        in_specs=[
            pl.BlockSpec((BT, 1), lambda i, be, nu: (i, 0)),
            pl.BlockSpec((N_TOKENS, D_MODEL), lambda i, be, nu: (0, 0)),
            pl.BlockSpec((1, HIDDEN, D_MODEL), lambda i, be, nu: (be[i], 0, 0)),
            pl.BlockSpec((1, 1, HIDDEN), lambda i, be, nu: (be[i], 0, 0)),
        ],
        out_specs=pl.BlockSpec((BT, HIDDEN), lambda i, be, nu: (i, 0)),
    )
    return pl.pallas_call(
        _ffn_up_body,
        grid_spec=grid_spec,
        out_shape=jax.ShapeDtypeStruct((PADT, HIDDEN), jnp.bfloat16),
        name="moe_ffn_up",
    )(be, nu, ssrc2, xb, w1, b1)


def _ffn_down_g_body(be_ref, nu_ref, h_ref, wg_ref, bg_ref, outg_ref):
    i = pl.program_id(0)

    @pl.when(i < nu_ref[0])
    def _():
        h32 = h_ref[...].astype(jnp.float32)
        g = lax.dot_general(h32, wg_ref[0], (((1,), (1,)), ((), ())),
                            preferred_element_type=jnp.float32)
        gg = g + bg_ref[0]
        outg_ref[...] = gg * (1.0 / (1.0 + jnp.exp(-gg)))


def _ffn_down_g(h_s, wg, bg, be, nu):
    grid_spec = pltpu.PrefetchScalarGridSpec(
        num_scalar_prefetch=2,
        grid=(NBLK,),
        in_specs=[
            pl.BlockSpec((BT, HIDDEN), lambda i, be, nu: (i, 0)),
            pl.BlockSpec((1, D_MODEL, HIDDEN), lambda i, be, nu: (be[i], 0, 0)),
            pl.BlockSpec((1, 1, D_MODEL), lambda i, be, nu: (be[i], 0, 0)),
        ],
        out_specs=pl.BlockSpec((BT, D_MODEL), lambda i, be, nu: (i, 0)),
    )
    return pl.pallas_call(
        _ffn_down_g_body,
        grid_spec=grid_spec,
        out_shape=jax.ShapeDtypeStruct((PADT, D_MODEL), jnp.float32),
        name="moe_ffn_down_g",
    )(be, nu, h_s, wg, bg)


def _ffn_down_v_body(be_ref, nu_ref, h_ref, wv_ref, bv_ref, outg_ref, sw_ref,
                     out_ref):
    i = pl.program_id(0)

    @pl.when(i < nu_ref[0])
    def _():
        h32 = h_ref[...].astype(jnp.float32)
        v = lax.dot_general(h32, wv_ref[0], (((1,), (1,)), ((), ())),
                            preferred_element_type=jnp.float32)
        vv = v + bv_ref[0]
        wcol = sw_ref[...].astype(jnp.float32)
        out_ref[...] = outg_ref[...] * vv * wcol


def _ffn_down_v(h_s, wv, bv, out_g, sw, be, nu):
    grid_spec = pltpu.PrefetchScalarGridSpec(
        num_scalar_prefetch=2,
        grid=(NBLK,),
        in_specs=[
            pl.BlockSpec((BT, HIDDEN), lambda i, be, nu: (i, 0)),
            pl.BlockSpec((1, D_MODEL, HIDDEN), lambda i, be, nu: (be[i], 0, 0)),
            pl.BlockSpec((1, 1, D_MODEL), lambda i, be, nu: (be[i], 0, 0)),
            pl.BlockSpec((BT, D_MODEL), lambda i, be, nu: (i, 0)),
            pl.BlockSpec((BT, 1), lambda i, be, nu: (i, 0)),
        ],
        out_specs=pl.BlockSpec((BT, D_MODEL), lambda i, be, nu: (i, 0)),
    )
    return pl.pallas_call(
        _ffn_down_v_body,
        grid_spec=grid_spec,
        out_shape=jax.ShapeDtypeStruct((PADT, D_MODEL), jnp.float32),
        name="moe_ffn_down_v",
    )(be, nu, h_s, wv, bv, out_g, sw)


def _combine(dest, out_s):
    tok_w = N_TOKENS // NW
    ck = 32
    mesh = plsc.VectorSubcoreMesh(core_axis_name="c", subcore_axis_name="s")

    @functools.partial(
        pl.kernel,
        out_type=jax.ShapeDtypeStruct((N_TOKENS, D_MODEL), jnp.float32),
        mesh=mesh,
        compiler_params=pltpu.CompilerParams(needs_layout_passes=False),
        scratch_types=[
            pltpu.VMEM((tok_w,), jnp.int32),
            pltpu.VMEM((tok_w,), jnp.int32),
            pltpu.VMEM((ck, D_MODEL), jnp.float32),
            pltpu.VMEM((ck, D_MODEL), jnp.float32),
            pltpu.SemaphoreType.DMA,
        ],
        name="moe_combine",
    )
    def k(dest_hbm, os_hbm, fin_hbm, d0_v, d1_v, r0, r1, sem):
        wid = lax.axis_index("s") * NUM_SC_CORES + lax.axis_index("c")
        base = wid * tok_w
        pltpu.sync_copy(dest_hbm.at[pl.ds(base, tok_w)], d0_v)
        pltpu.sync_copy(dest_hbm.at[pl.ds(N_TOKENS + base, tok_w)], d1_v)
        for c in range(tok_w // ck):
            pltpu.async_copy(os_hbm.at[d0_v.at[pl.ds(c * ck, ck)]], r0,
                             sem).wait()
            pltpu.async_copy(os_hbm.at[d1_v.at[pl.ds(c * ck, ck)]], r1,
                             sem).wait()

            def abody(r, _):
                for l in range(D_MODEL // 16):
                    lsl = pl.ds(l * 16, 16)
                    r0[r, lsl] = r0[r, lsl] + r1[r, lsl]
                return 0

            lax.fori_loop(0, ck, abody, 0)
            pltpu.sync_copy(r0, fin_hbm.at[pl.ds(base + c * ck, ck)])

    return k(dest.reshape(N_PAIRS), out_s)


def kernel(x, task_emb, base_gate_w, lora_A, lora_B, W1, b1, Wg, bg, Wv, bv,
           task_id_tensor):
    bsz, seqlen, dim = x.shape
    x_flat = x.reshape(-1, dim)
    tids2 = task_id_tensor.reshape(-1, 1).astype(jnp.int32)
    xb, dest, wp, be, nu = _gate(x_flat, tids2, task_emb, base_gate_w,
                                 lora_A, lora_B)
    ssrc, sw = _sort_scatter(dest, wp.reshape(N_PAIRS))
    ben, nun = be.reshape(NBLK), nu.reshape(1)
    h_s = _ffn_up(ssrc.reshape(PADT, 1), xb, W1, b1[:, None, :], ben, nun)
    out_g = _ffn_down_g(h_s, Wg, bg[:, None, :], ben, nun)
    out_s = _ffn_down_v(h_s, Wv, bv[:, None, :], out_g,
                        sw.reshape(PADT, 1), ben, nun)
    fin = _combine(dest, out_s)
    return fin.reshape(bsz, seqlen, dim)
